# baseline (speedup 1.0000x reference)
import functools

import jax
import jax.numpy as jnp
from jax import lax
from jax.experimental import pallas as pl
from jax.experimental.pallas import tpu as pltpu
from jax.experimental.pallas import tpu_sc as plsc

HIDDEN = 5120
NUM_EXPERTS = 128
TOP_K = 4
MIN_TEMP = 0.1
EPS = 1e-08

BLK = 1024
HALF = BLK // 2

N_ROWS = 8192
N_WORKERS = 32
ROWS_PER_W = N_ROWS // N_WORKERS
GROUPS = ROWS_PER_W // 16
W_PER_BLK = BLK // ROWS_PER_W


def _softmax_kernel(t_ref, xa_ref, xb_ref, w_ref, s_ref):
    inv_t = 1.0 / t_ref[0]
    wb = w_ref[...].astype(jnp.bfloat16)
    for h, x_ref in enumerate((xa_ref, xb_ref)):
        xh = x_ref[...].astype(jnp.bfloat16)
        logits = jax.lax.dot_general(
            xh, wb,
            dimension_numbers=(((1,), (1,)), ((), ())),
            preferred_element_type=jnp.float32,
        )
        ls = logits * inv_t
        m = jnp.max(ls, axis=-1, keepdims=True)
        e = jnp.exp(ls - m)
        denom = jnp.sum(e, axis=-1, keepdims=True)
        s = e / denom
        for q in range(HALF // ROWS_PER_W):
            sub = s[q * ROWS_PER_W:(q + 1) * ROWS_PER_W, :]
            s_ref[2 * h + q] = sub.T


def _tc_softmax(t, x, gate_w):
    n_rows = x.shape[0]
    return pl.pallas_call(
        _softmax_kernel,
        grid=(n_rows // BLK,),
        in_specs=[
            pl.BlockSpec(memory_space=pltpu.SMEM),
            pl.BlockSpec((HALF, HIDDEN), lambda i: (2 * i, 0)),
            pl.BlockSpec((HALF, HIDDEN), lambda i: (2 * i + 1, 0)),
            pl.BlockSpec((NUM_EXPERTS, HIDDEN), lambda i: (0, 0)),
        ],
        out_specs=pl.BlockSpec(
            (W_PER_BLK, NUM_EXPERTS, ROWS_PER_W), lambda i: (i, 0, 0)
        ),
        out_shape=jax.ShapeDtypeStruct(
            (N_WORKERS, NUM_EXPERTS, ROWS_PER_W), jnp.float32
        ),
    )(t, x, x, gate_w)


def _sc_top4_body(s_hbm, idx_hbm, wt_hbm, buf, idx_v, wt_v):
    wid = lax.axis_index("s") * 2 + lax.axis_index("c")
    pltpu.sync_copy(s_hbm.at[wid], buf)

    def group(g, carry):
        sl = pl.ds(g * 16, 16)
        vals = []
        picks = []
        for _k in range(TOP_K):
            mx = jnp.full((16,), -1.0, jnp.float32)
            pk = jnp.zeros((16,), jnp.int32)
            for c in range(NUM_EXPERTS):
                col = jnp.full((16,), c, jnp.int32)
                sc = buf[c, sl]
                gt = sc > mx
                for prev in picks:
                    gt = gt & (col != prev)
                mx = jnp.where(gt, sc, mx)
                pk = jnp.where(gt, col, pk)
            vals.append(mx)
            picks.append(pk)
        total = vals[0] + vals[1] + vals[2] + vals[3] + EPS
        for _k in range(TOP_K):
            idx_v[_k, sl] = picks[_k]
            wt_v[_k, sl] = vals[_k] / total
        return carry

    lax.fori_loop(0, GROUPS, group, 0)
    pltpu.sync_copy(idx_v, idx_hbm.at[wid])
    pltpu.sync_copy(wt_v, wt_hbm.at[wid])


@functools.partial(jax.jit, static_argnames=())
def kernel(x, gate_w, temperature):
    t = jnp.maximum(jax.nn.softplus(temperature), MIN_TEMP).reshape((1,))
    scores = _tc_softmax(t, x, gate_w)

    mesh = plsc.VectorSubcoreMesh(core_axis_name="c", subcore_axis_name="s")
    sc_top4 = functools.partial(
        pl.kernel,
        mesh=mesh,
        out_type=[
            jax.ShapeDtypeStruct((N_WORKERS, TOP_K, ROWS_PER_W), jnp.int32),
            jax.ShapeDtypeStruct((N_WORKERS, TOP_K, ROWS_PER_W), jnp.float32),
        ],
        scratch_types=[
            pltpu.VMEM((NUM_EXPERTS, ROWS_PER_W), jnp.float32),
            pltpu.VMEM((TOP_K, ROWS_PER_W), jnp.int32),
            pltpu.VMEM((TOP_K, ROWS_PER_W), jnp.float32),
        ],
    )(_sc_top4_body)
    idx3, wt3 = sc_top4(scores)
    idx = jnp.transpose(idx3, (0, 2, 1)).reshape(N_ROWS, TOP_K)
    wt = jnp.transpose(wt3, (0, 2, 1)).reshape(N_ROWS, TOP_K)
    return idx, wt


# speedup vs baseline: 2.3646x; 2.3646x over previous
import functools

import jax
import jax.numpy as jnp
from jax.experimental import pallas as pl
from jax.experimental.pallas import tpu as pltpu

HIDDEN = 5120
NUM_EXPERTS = 128
TOP_K = 4
MIN_TEMP = 0.1
EPS = 1e-08

BLK = 1024
HALF = BLK // 2


def _top4(s, iota):
    idxs = []
    vals = []
    for _ in range(TOP_K):
        mx = jnp.max(s, axis=-1, keepdims=True)
        pick = jnp.min(
            jnp.where(s == mx, iota, NUM_EXPERTS), axis=-1, keepdims=True
        )
        vals.append(mx)
        idxs.append(pick)
        s = jnp.where(iota == pick, -1.0, s)
    total = vals[0] + vals[1] + vals[2] + vals[3] + EPS
    idx = jnp.concatenate(idxs, axis=1).T
    wt = (jnp.concatenate(vals, axis=1) / total).T
    return idx, wt


def _gate_kernel(t_ref, xa_ref, xb_ref, w_ref, idx_ref, wt_ref):
    inv_t = 1.0 / t_ref[0]
    wb = w_ref[...].astype(jnp.bfloat16)
    iota = jax.lax.broadcasted_iota(jnp.int32, (HALF, NUM_EXPERTS), 1)
    for h, x_ref in enumerate((xa_ref, xb_ref)):
        xh = x_ref[...].astype(jnp.bfloat16)
        logits = jax.lax.dot_general(
            xh, wb,
            dimension_numbers=(((1,), (1,)), ((), ())),
            preferred_element_type=jnp.float32,
        )
        ls = logits * inv_t
        m = jnp.max(ls, axis=-1, keepdims=True)
        e = jnp.exp(ls - m)
        denom = jnp.sum(e, axis=-1, keepdims=True)
        s = e / denom
        idx, wt = _top4(s, iota)
        sl = pl.ds(h * HALF, HALF)
        idx_ref[:, sl] = idx
        wt_ref[:, sl] = wt


@functools.partial(jax.jit, static_argnames=())
def kernel(x, gate_w, temperature):
    n_rows = x.shape[0]
    t = jnp.maximum(jax.nn.softplus(temperature), MIN_TEMP).reshape((1,))
    grid = (n_rows // BLK,)
    idx, w = pl.pallas_call(
        _gate_kernel,
        grid=grid,
        in_specs=[
            pl.BlockSpec(memory_space=pltpu.SMEM),
            pl.BlockSpec((HALF, HIDDEN), lambda i: (2 * i, 0)),
            pl.BlockSpec((HALF, HIDDEN), lambda i: (2 * i + 1, 0)),
            pl.BlockSpec((NUM_EXPERTS, HIDDEN), lambda i: (0, 0)),
        ],
        out_specs=[
            pl.BlockSpec((TOP_K, BLK), lambda i: (0, i)),
            pl.BlockSpec((TOP_K, BLK), lambda i: (0, i)),
        ],
        out_shape=[
            jax.ShapeDtypeStruct((TOP_K, n_rows), jnp.int32),
            jax.ShapeDtypeStruct((TOP_K, n_rows), jnp.float32),
        ],
    )(t, x, x, gate_w)
    return idx.T, w.T
